# baseline (speedup 1.0000x reference)
import functools

import jax
import jax.numpy as jnp
from jax import lax
from jax.experimental import pallas as pl
from jax.experimental.pallas import tpu as pltpu
from jax.experimental.pallas import tpu_sc as plsc

NUM_CLASS = 100000
BATCH = 1024
CTX_DIM = 512
N_CLS_CTX = 4
PREFIX_LEN = 5
SUFFIX_LEN = 68
CLIP_LEN = 77

ROW = CLIP_LEN * CTX_DIM
PRE = PREFIX_LEN * CTX_DIM
MID = N_CLS_CTX * CTX_DIM
SUF = SUFFIX_LEN * CTX_DIM

NC, NS = 2, 16
NW = NC * NS
BPW = BATCH // NW
HALF = BPW // 2

_mesh = plsc.VectorSubcoreMesh(core_axis_name="c", subcore_axis_name="s")


@functools.partial(
    pl.kernel,
    mesh=_mesh,
    out_type=jax.ShapeDtypeStruct((BATCH, ROW), jnp.float32),
    scratch_types=[
        pltpu.VMEM((BPW,), jnp.int32),
        pltpu.VMEM((BPW, MID), jnp.float32),
        pltpu.VMEM_SHARED((HALF, ROW), jnp.float32),
        pltpu.SemaphoreType.DMA,
        pltpu.SemaphoreType.DMA,
    ],
)
def _prompt_kernel(label_hbm, table_hbm, prefix_hbm, suffix_hbm, out_hbm,
                   idx_v, rows_v, pattern_s, gsem, psem):
    sid = lax.axis_index("s")
    wid = sid * NC + lax.axis_index("c")
    base = wid * BPW

    pltpu.sync_copy(prefix_hbm, pattern_s.at[sid, pl.ds(0, PRE)])
    pltpu.sync_copy(suffix_hbm, pattern_s.at[sid, pl.ds(PRE + MID, SUF)])

    pltpu.sync_copy(label_hbm.at[pl.ds(base, BPW)], idx_v)
    gcp = pltpu.async_copy(table_hbm.at[idx_v], rows_v, gsem)

    plsc.subcore_barrier()

    p0 = pltpu.async_copy(pattern_s, out_hbm.at[pl.ds(base, HALF)], psem)
    p1 = pltpu.async_copy(pattern_s, out_hbm.at[pl.ds(base + HALF, HALF)], psem)
    gcp.wait()

    p0.wait()
    pltpu.sync_copy(rows_v.at[pl.ds(0, HALF)],
                    out_hbm.at[pl.ds(base, HALF), pl.ds(PRE, MID)])
    p1.wait()
    pltpu.sync_copy(rows_v.at[pl.ds(HALF, HALF)],
                    out_hbm.at[pl.ds(base + HALF, HALF), pl.ds(PRE, MID)])


def kernel(label, cls_ctx, token_prefix, token_suffix):
    table = cls_ctx.reshape(NUM_CLASS, MID)
    pre = token_prefix.reshape(PRE)
    suf = token_suffix.reshape(SUF)
    out = _prompt_kernel(label.astype(jnp.int32), table, pre, suf)
    return out.reshape(BATCH, CLIP_LEN, CTX_DIM)


# speedup vs baseline: 4.0921x; 4.0921x over previous
import functools

import jax
import jax.numpy as jnp
from jax import lax
from jax.experimental import pallas as pl
from jax.experimental.pallas import tpu as pltpu
from jax.experimental.pallas import tpu_sc as plsc

NUM_CLASS = 100000
BATCH = 1024
CTX_DIM = 512
N_CLS_CTX = 4
PREFIX_LEN = 5
SUFFIX_LEN = 68
CLIP_LEN = 77

NC, NS = 2, 16
NW = NC * NS
BPW = BATCH // NW

_mesh = plsc.VectorSubcoreMesh(core_axis_name="c", subcore_axis_name="s")


@functools.partial(
    pl.kernel,
    mesh=_mesh,
    out_type=jax.ShapeDtypeStruct((BATCH, N_CLS_CTX, CTX_DIM), jnp.float32),
    scratch_types=[
        pltpu.VMEM((BPW,), jnp.int32),
        pltpu.VMEM((BPW, N_CLS_CTX, CTX_DIM), jnp.float32),
        pltpu.SemaphoreType.DMA,
    ],
    compiler_params=pltpu.CompilerParams(use_tc_tiling_on_sc=True),
)
def _sc_gather(label_hbm, table_hbm, out_hbm, idx_v, rows_v, gsem):
    wid = lax.axis_index("s") * NC + lax.axis_index("c")
    base = wid * BPW
    pltpu.sync_copy(label_hbm.at[pl.ds(base, BPW)], idx_v)
    pltpu.async_copy(table_hbm.at[idx_v], rows_v, gsem).wait()
    pltpu.sync_copy(rows_v, out_hbm.at[pl.ds(base, BPW)])


BB = 8


def _tc_assemble_body(g_ref, p_ref, s_ref, o_ref):
    o_ref[:, 0:PREFIX_LEN, :] = jnp.broadcast_to(
        p_ref[...], (BB, PREFIX_LEN, CTX_DIM))
    o_ref[:, PREFIX_LEN:PREFIX_LEN + N_CLS_CTX, :] = g_ref[...]
    o_ref[:, PREFIX_LEN + N_CLS_CTX:CLIP_LEN, :] = jnp.broadcast_to(
        s_ref[...], (BB, SUFFIX_LEN, CTX_DIM))


def _tc_assemble(gathered, token_prefix, token_suffix):
    return pl.pallas_call(
        _tc_assemble_body,
        grid=(BATCH // BB,),
        in_specs=[
            pl.BlockSpec((BB, N_CLS_CTX, CTX_DIM), lambda i: (i, 0, 0)),
            pl.BlockSpec((1, PREFIX_LEN, CTX_DIM), lambda i: (0, 0, 0)),
            pl.BlockSpec((1, SUFFIX_LEN, CTX_DIM), lambda i: (0, 0, 0)),
        ],
        out_specs=pl.BlockSpec((BB, CLIP_LEN, CTX_DIM), lambda i: (i, 0, 0)),
        out_shape=jax.ShapeDtypeStruct((BATCH, CLIP_LEN, CTX_DIM), jnp.float32),
    )(gathered, token_prefix, token_suffix)


def kernel(label, cls_ctx, token_prefix, token_suffix):
    gathered = _sc_gather(label.astype(jnp.int32), cls_ctx)
    return _tc_assemble(gathered, token_prefix, token_suffix)


# speedup vs baseline: 4.7874x; 1.1699x over previous
import functools

import jax
import jax.numpy as jnp
from jax import lax
from jax.experimental import pallas as pl
from jax.experimental.pallas import tpu as pltpu
from jax.experimental.pallas import tpu_sc as plsc

NUM_CLASS = 100000
BATCH = 1024
CTX_DIM = 512
N_CLS_CTX = 4
PREFIX_LEN = 5
SUFFIX_LEN = 68
CLIP_LEN = 77
MID_START = PREFIX_LEN
SUF_START = PREFIX_LEN + N_CLS_CTX
HEAD = 16
TAIL = CLIP_LEN - HEAD
SUF_HEAD = HEAD - SUF_START

NC, NS = 2, 16
NW = NC * NS
BPW = BATCH // NW
CHUNK = 8
NCHUNK = BPW // CHUNK
LANES = 16
NKC = CTX_DIM // LANES

_mesh = plsc.VectorSubcoreMesh(core_axis_name="c", subcore_axis_name="s")


@functools.partial(
    pl.kernel,
    mesh=_mesh,
    out_type=jax.ShapeDtypeStruct((BATCH, CLIP_LEN, CTX_DIM), jnp.float32),
    scratch_types=[
        pltpu.VMEM((BPW,), jnp.int32),
        pltpu.VMEM((CHUNK, N_CLS_CTX, CTX_DIM), jnp.float32),
        pltpu.VMEM((CHUNK, N_CLS_CTX, CTX_DIM), jnp.float32),
        pltpu.VMEM((HEAD, CTX_DIM), jnp.float32),
        pltpu.VMEM((HEAD, CTX_DIM), jnp.float32),
        pltpu.VMEM((PREFIX_LEN, CTX_DIM), jnp.float32),
        pltpu.VMEM((SUFFIX_LEN, CTX_DIM), jnp.float32),
        pltpu.VMEM((TAIL, CTX_DIM), jnp.float32),
        pltpu.SemaphoreType.DMA,
        pltpu.SemaphoreType.DMA,
        pltpu.SemaphoreType.DMA,
        pltpu.SemaphoreType.DMA,
        pltpu.SemaphoreType.DMA,
    ],
    compiler_params=pltpu.CompilerParams(use_tc_tiling_on_sc=True),
)
def _prompt_kernel(label_hbm, table_hbm, prefix_hbm, suffix_hbm, out_hbm,
                   idx_v, rows_a, rows_b, tmpl_a, tmpl_b, pre_v, suf_v,
                   tail_v, gsem_a, gsem_b, ssem, tsem_a, tsem_b):
    wid = lax.axis_index("s") * NC + lax.axis_index("c")
    base = wid * BPW
    rows_bufs = (rows_a, rows_b)
    gsems = (gsem_a, gsem_b)
    tmpls = (tmpl_a, tmpl_b)
    tsems = (tsem_a, tsem_b)

    pltpu.sync_copy(label_hbm.at[pl.ds(base, BPW)], idx_v)
    g0 = pltpu.async_copy(table_hbm.at[idx_v.at[pl.ds(0, CHUNK)]], rows_a,
                          gsem_a)
    g1 = pltpu.async_copy(table_hbm.at[idx_v.at[pl.ds(CHUNK, CHUNK)]], rows_b,
                          gsem_b)

    pltpu.sync_copy(prefix_hbm.at[0], pre_v)
    pltpu.sync_copy(suffix_hbm.at[0], suf_v)
    for tmpl in (tmpl_a, tmpl_b):
        for t in range(PREFIX_LEN):
            for k in range(NKC):
                sl = pl.ds(k * LANES, LANES)
                tmpl[t, sl] = pre_v[t, sl]
        for t in range(SUF_HEAD):
            for k in range(NKC):
                sl = pl.ds(k * LANES, LANES)
                tmpl[SUF_START + t, sl] = suf_v[t, sl]

    def shift_row(t, _):
        for k in range(NKC):
            sl = pl.ds(k * LANES, LANES)
            tail_v[t, sl] = suf_v[t + SUF_HEAD, sl]
        return ()
    lax.fori_loop(0, TAIL, shift_row, ())

    g0.wait()
    g1.wait()

    for c in range(NCHUNK):
        rows_v = rows_bufs[c % 2]
        gsem = gsems[c % 2]
        if c >= 2:
            pltpu.make_async_copy(
                table_hbm.at[idx_v.at[pl.ds(0, CHUNK)]], rows_v, gsem).wait()

        def row_pair(q, _, _c=c, _rows=rows_v):
            for b in (0, 1):
                j = _c * CHUNK + 2 * q + b
                row = base + j
                tmpl = tmpls[b]
                tsem = tsems[b]
                if _c == 0:
                    @pl.when(q >= 1)
                    def _():
                        pltpu.make_async_copy(
                            tmpl, out_hbm.at[row, pl.ds(0, HEAD)],
                            tsem).wait()
                else:
                    pltpu.make_async_copy(
                        tmpl, out_hbm.at[row, pl.ds(0, HEAD)], tsem).wait()
                pltpu.async_copy(tail_v, out_hbm.at[row, pl.ds(HEAD, TAIL)],
                                 ssem)
                r = 2 * q + b
                for m in range(N_CLS_CTX):
                    for k in range(NKC):
                        sl = pl.ds(k * LANES, LANES)
                        tmpl[MID_START + m, sl] = _rows[r, m, sl]
                pltpu.async_copy(tmpl, out_hbm.at[row, pl.ds(0, HEAD)], tsem)
            return ()
        lax.fori_loop(0, CHUNK // 2, row_pair, ())

        if c + 2 < NCHUNK:
            nxt = (c + 2) * CHUNK
            pltpu.async_copy(
                table_hbm.at[idx_v.at[pl.ds(nxt, CHUNK)]], rows_v, gsem)

    for _ in range(BPW):
        pltpu.make_async_copy(tail_v, out_hbm.at[base, pl.ds(HEAD, TAIL)],
                              ssem).wait()
    pltpu.make_async_copy(tmpl_a, out_hbm.at[base, pl.ds(0, HEAD)],
                          tsem_a).wait()
    pltpu.make_async_copy(tmpl_b, out_hbm.at[base, pl.ds(0, HEAD)],
                          tsem_b).wait()


def kernel(label, cls_ctx, token_prefix, token_suffix):
    return _prompt_kernel(label.astype(jnp.int32), cls_ctx,
                          token_prefix, token_suffix)
